# baseline (speedup 1.0000x reference)
import functools
import math

import jax
import jax.numpy as jnp
from jax.experimental import pallas as pl

NEG_INF = float("-inf")
F32 = jnp.float32


BF16 = jnp.bfloat16


def _dot(a, b):
    return jnp.dot(a.astype(BF16), b.astype(BF16), preferred_element_type=F32)


def _pre_kernel(x_ref, iW1, ib1, iW2, ib2, fW1, fb1, fW2, fb2, mW, mb,
                int_ref, feat_ref, msg_ref):
    x = x_ref[...]
    h = jax.nn.gelu(_dot(x, iW1[...]) + ib1[...])
    int_ref[...] = _dot(h, iW2[...]) + ib2[...]
    h = jax.nn.gelu(_dot(x, fW1[...]) + fb1[...])
    feat_ref[...] = _dot(h, fW2[...]) + fb2[...]
    msg_ref[...] = _dot(x, mW[...]) + mb[...]


def _pre_call(cur, p, mp, bm):
    S, D = cur.shape
    H = D // 2
    full = lambda shp: pl.BlockSpec(shp, lambda i: (0,) * len(shp))
    row = pl.BlockSpec((bm, D), lambda i: (i, 0))
    return pl.pallas_call(
        _pre_kernel,
        grid=(S // bm,),
        in_specs=[row,
                  full((D, H)), full((1, H)), full((H, D)), full((1, D)),
                  full((D, H)), full((1, H)), full((H, D)), full((1, D)),
                  full((D, D)), full((1, D))],
        out_specs=(row, row, row),
        out_shape=(jax.ShapeDtypeStruct((S, D), F32),) * 3,
    )(cur, p["iW1"], p["ib1"].reshape(1, H), p["iW2"], p["ib2"].reshape(1, D),
      p["fW1"], p["fb1"].reshape(1, H), p["fW2"], p["fb2"].reshape(1, D),
      mp["mW"], mp["mb"].reshape(1, D))


def _attn_kernel(kk, scale, int_ref, feat_ref, msg_ref, lmask_ref, out_ref):
    s = jax.lax.dot_general(int_ref[...].astype(BF16), feat_ref[...].astype(BF16),
                            (((1,), (1,)), ((), ())),
                            preferred_element_type=F32)
    s = s * scale + lmask_ref[...]
    rowmax = jnp.max(s, axis=-1, keepdims=True)

    def body(_, t):
        m = jnp.max(t, axis=-1, keepdims=True)
        return jnp.where(t >= m, NEG_INF, t)

    t = jax.lax.fori_loop(0, kk - 1, body, s)
    thr = jnp.max(t, axis=-1, keepdims=True)
    w = jnp.where(s >= thr, jnp.exp(s - rowmax), 0.0)
    p = w / jnp.sum(w, axis=-1, keepdims=True)
    out_ref[...] = _dot(p, msg_ref[...])


def _attn_call(intents, feats, msgs, lmask, kk, bm):
    S, D = intents.shape
    scale = 1.0 / math.sqrt(D)
    full = lambda shp: pl.BlockSpec(shp, lambda i: (0,) * len(shp))
    row = pl.BlockSpec((bm, D), lambda i: (i, 0))
    return pl.pallas_call(
        functools.partial(_attn_kernel, kk, scale),
        grid=(S // bm,),
        in_specs=[row, full((S, D)), full((S, D)),
                  pl.BlockSpec((bm, S), lambda i: (i, 0))],
        out_specs=row,
        out_shape=jax.ShapeDtypeStruct((S, D), F32),
    )(intents, feats, msgs, lmask)


def _gate_kernel(x_ref, agg_ref, aW, ab, gWx, gWa, gb, oW, ob, out_ref):
    x = x_ref[...]
    a2 = _dot(agg_ref[...], aW[...]) + ab[...]
    g = jax.nn.sigmoid(_dot(x, gWx[...]) + _dot(a2, gWa[...]) + gb[...])
    u = x * (1.0 - g) + a2 * g
    out_ref[...] = _dot(u, oW[...]) + ob[...]


def _gate_call(cur, agg, mp, bm):
    S, D = cur.shape
    full = lambda shp: pl.BlockSpec(shp, lambda i: (0,) * len(shp))
    row = pl.BlockSpec((bm, D), lambda i: (i, 0))
    return pl.pallas_call(
        _gate_kernel,
        grid=(S // bm,),
        in_specs=[row, row,
                  full((D, D)), full((1, D)),
                  full((D, D)), full((D, D)), full((1, D)),
                  full((D, D)), full((1, D))],
        out_specs=row,
        out_shape=jax.ShapeDtypeStruct((S, D), F32),
    )(cur, agg, mp["aW"], mp["ab"].reshape(1, D),
      mp["gW"][:D], mp["gW"][D:], mp["gb"].reshape(1, D),
      mp["oW"], mp["ob"].reshape(1, D))


def _blockagg_kernel(xr_ref, W, b, out_ref):
    m = jnp.mean(xr_ref[...], axis=1)
    out_ref[...] = jax.nn.gelu(_dot(m, W[...]) + b[...])


def _blockagg_call(xr, W, b, bm):
    nb, bs, D = xr.shape
    full = lambda shp: pl.BlockSpec(shp, lambda i: (0,) * len(shp))
    return pl.pallas_call(
        _blockagg_kernel,
        grid=(nb // bm,),
        in_specs=[pl.BlockSpec((bm, bs, D), lambda i: (i, 0, 0)),
                  full((D, D)), full((1, D))],
        out_specs=pl.BlockSpec((bm, D), lambda i: (i, 0)),
        out_shape=jax.ShapeDtypeStruct((nb, D), F32),
    )(xr, W, b.reshape(1, D))


def _linear_kernel(x_ref, W, b, out_ref):
    out_ref[...] = _dot(x_ref[...], W[...]) + b[...]


def _linear_call(x, W, b, bm):
    S, D = x.shape
    full = lambda shp: pl.BlockSpec(shp, lambda i: (0,) * len(shp))
    row = pl.BlockSpec((bm, D), lambda i: (i, 0))
    return pl.pallas_call(
        _linear_kernel,
        grid=(S // bm,),
        in_specs=[row, full((D, D)), full((1, D))],
        out_specs=row,
        out_shape=jax.ShapeDtypeStruct((S, D), F32),
    )(x, W, b.reshape(1, D))


def _final_kernel(o0_ref, d1_ref, d2_ref, F0, F1, F2, fb, out_ref):
    acc = _dot(o0_ref[...], F0[...])
    acc = acc + _dot(d1_ref[...], F1[...])
    acc = acc + _dot(d2_ref[...], F2[...])
    out_ref[...] = acc + fb[...]


def _final_call(o0, d1, d2, fW, fb, bm):
    S, D = o0.shape
    full = lambda shp: pl.BlockSpec(shp, lambda i: (0,) * len(shp))
    row = pl.BlockSpec((bm, D), lambda i: (i, 0))
    return pl.pallas_call(
        _final_kernel,
        grid=(S // bm,),
        in_specs=[row, row, row,
                  full((D, D)), full((D, D)), full((D, D)), full((1, D))],
        out_specs=row,
        out_shape=jax.ShapeDtypeStruct((S, D), F32),
    )(o0, d1, d2, fW[:D], fW[D:2 * D], fW[2 * D:], fb.reshape(1, D))


def _locality_mask(S, lb):
    L = lb.shape[-1]
    t = jnp.arange(2 * S)
    relv = jnp.where(t >= S, t - 2 * S, t)
    idx = jnp.clip(relv, -(L // 2), L // 2 - 1) + L // 2
    v = jax.nn.one_hot(idx, L, dtype=F32) @ lb
    v = jnp.where(relv > 0, NEG_INF, v)
    X = jnp.broadcast_to(v, (S, 2 * S)).reshape(-1)[: S * (2 * S - 1)]
    return X.reshape(S, 2 * S - 1)[:, :S]


BLOCK_SIZES_ = (1, 4, 16)


def kernel(x, params):
    xb = x[0]
    S, D = xb.shape
    outs = []
    for level, bs in enumerate(BLOCK_SIZES_):
        if bs == 1:
            cur = xb
        else:
            ap = params["block_agg"][level - 1]
            cur = _blockagg_call(xb.reshape(S // bs, bs, D), ap["W"], ap["b"],
                                 bm=min(256, S // bs))
        Sl = cur.shape[0]
        bm = min(256, Sl)
        p = params["pred"][level]
        intents, feats, msgs = _pre_call(cur, p, params["mp"][level], bm)
        lmask = _locality_mask(Sl, p["lb"])
        agg = _attn_call(intents, feats, msgs, lmask, 32 // (2 ** level), bm)
        lo = _gate_call(cur, agg, params["mp"][level], bm)
        if bs > 1:
            dp = params["block_dist"][level - 1]
            dist = _linear_call(lo, dp["W"], dp["b"], bm)
            outs.append(jnp.repeat(dist, bs, axis=0))
        else:
            outs.append(lo)
    out = _final_call(outs[0], outs[1], outs[2], params["fW"], params["fb"],
                      bm=256)
    return out[None]


# speedup vs baseline: 1.3418x; 1.3418x over previous
import functools
import math

import jax
import jax.numpy as jnp
from jax.experimental import pallas as pl

NEG_INF = float("-inf")
F32 = jnp.float32


BF16 = jnp.bfloat16


def _dot(a, b):
    return jnp.dot(a.astype(BF16), b.astype(BF16), preferred_element_type=F32)


def _dotf(a, b):
    return jnp.dot(a, b, preferred_element_type=F32)


def _pre_kernel(x_ref, iW1, ib1, iW2, ib2, fW1, fb1, fW2, fb2, mW, mb,
                int_ref, feat_ref, msg_ref):
    x = x_ref[...]
    h = jax.nn.gelu(_dotf(x, iW1[...]) + ib1[...])
    int_ref[...] = _dotf(h, iW2[...]) + ib2[...]
    h = jax.nn.gelu(_dotf(x, fW1[...]) + fb1[...])
    feat_ref[...] = _dotf(h, fW2[...]) + fb2[...]
    msg_ref[...] = _dot(x, mW[...]) + mb[...]


def _pre_call(cur, p, mp, bm):
    S, D = cur.shape
    H = D // 2
    full = lambda shp: pl.BlockSpec(shp, lambda i: (0,) * len(shp))
    row = pl.BlockSpec((bm, D), lambda i: (i, 0))
    return pl.pallas_call(
        _pre_kernel,
        grid=(S // bm,),
        in_specs=[row,
                  full((D, H)), full((1, H)), full((H, D)), full((1, D)),
                  full((D, H)), full((1, H)), full((H, D)), full((1, D)),
                  full((D, D)), full((1, D))],
        out_specs=(row, row, row),
        out_shape=(jax.ShapeDtypeStruct((S, D), F32),) * 3,
    )(cur, p["iW1"], p["ib1"].reshape(1, H), p["iW2"], p["ib2"].reshape(1, D),
      p["fW1"], p["fb1"].reshape(1, H), p["fW2"], p["fb2"].reshape(1, D),
      mp["mW"], mp["mb"].reshape(1, D))


def _attn_kernel(kk, scale, i0, int_ref, feat_ref, msg_ref, band_ref, out_ref):
    s = jax.lax.dot_general(int_ref[...], feat_ref[...],
                            (((1,), (1,)), ((), ())),
                            preferred_element_type=F32) * scale
    BM, W = s.shape
    BW = band_ref.shape[1]
    lo = max(0, i0 - (BW - BM))
    dlo = lo - (i0 - (BW - BM))
    sb = s[:, lo:] + band_ref[:, dlo:]
    s = jnp.concatenate([s[:, :lo], sb], axis=1) if lo > 0 else sb
    rows = i0 + jax.lax.broadcasted_iota(jnp.int32, (BM, W), 0)
    cols = jax.lax.broadcasted_iota(jnp.int32, (BM, W), 1)
    s = jnp.where(cols > rows, NEG_INF, s)
    rowmax = jnp.max(s, axis=-1, keepdims=True)

    def body(_, t):
        m = jnp.max(t, axis=-1, keepdims=True)
        return jnp.where(t >= m, NEG_INF, t)

    t = jax.lax.fori_loop(0, kk - 1, body, s)
    thr = jnp.max(t, axis=-1, keepdims=True)
    w = jnp.where(s >= thr, jnp.exp(s - rowmax), 0.0)
    p = w / jnp.sum(w, axis=-1, keepdims=True)
    out_ref[...] = jnp.dot(p, msg_ref[...], preferred_element_type=F32)


def _attn_call(intents, feats, msgs, band, kk, bm):
    S, D = intents.shape
    scale = 1.0 / math.sqrt(D)
    outs = []
    for i in range(S // bm):
        W = (i + 1) * bm
        out = pl.pallas_call(
            functools.partial(_attn_kernel, kk, scale, i * bm),
            grid=(1,),
            in_specs=[pl.BlockSpec((bm, D), lambda g, i=i: (i, 0)),
                      pl.BlockSpec((W, D), lambda g: (0, 0)),
                      pl.BlockSpec((W, D), lambda g: (0, 0)),
                      pl.BlockSpec(band.shape, lambda g: (0, 0))],
            out_specs=pl.BlockSpec((bm, D), lambda g: (0, 0)),
            out_shape=jax.ShapeDtypeStruct((bm, D), F32),
        )(intents, feats, msgs, band)
        outs.append(out)
    return jnp.concatenate(outs, axis=0)


def _gate_kernel(x_ref, agg_ref, aW, ab, gWx, gWa, gb, oW, ob, out_ref):
    x = x_ref[...]
    a2 = _dot(agg_ref[...], aW[...]) + ab[...]
    g = jax.nn.sigmoid(_dot(x, gWx[...]) + _dot(a2, gWa[...]) + gb[...])
    u = x * (1.0 - g) + a2 * g
    out_ref[...] = _dot(u, oW[...]) + ob[...]


def _gate_call(cur, agg, mp, bm):
    S, D = cur.shape
    full = lambda shp: pl.BlockSpec(shp, lambda i: (0,) * len(shp))
    row = pl.BlockSpec((bm, D), lambda i: (i, 0))
    return pl.pallas_call(
        _gate_kernel,
        grid=(S // bm,),
        in_specs=[row, row,
                  full((D, D)), full((1, D)),
                  full((D, D)), full((D, D)), full((1, D)),
                  full((D, D)), full((1, D))],
        out_specs=row,
        out_shape=jax.ShapeDtypeStruct((S, D), F32),
    )(cur, agg, mp["aW"], mp["ab"].reshape(1, D),
      mp["gW"][:D], mp["gW"][D:], mp["gb"].reshape(1, D),
      mp["oW"], mp["ob"].reshape(1, D))


def _blockagg_kernel(xr_ref, W, b, out_ref):
    m = jnp.mean(xr_ref[...], axis=1)
    out_ref[...] = jax.nn.gelu(_dot(m, W[...]) + b[...])


def _blockagg_call(xr, W, b, bm):
    nb, bs, D = xr.shape
    full = lambda shp: pl.BlockSpec(shp, lambda i: (0,) * len(shp))
    return pl.pallas_call(
        _blockagg_kernel,
        grid=(nb // bm,),
        in_specs=[pl.BlockSpec((bm, bs, D), lambda i: (i, 0, 0)),
                  full((D, D)), full((1, D))],
        out_specs=pl.BlockSpec((bm, D), lambda i: (i, 0)),
        out_shape=jax.ShapeDtypeStruct((nb, D), F32),
    )(xr, W, b.reshape(1, D))


def _linear_kernel(x_ref, W, b, out_ref):
    out_ref[...] = _dot(x_ref[...], W[...]) + b[...]


def _linear_call(x, W, b, bm):
    S, D = x.shape
    full = lambda shp: pl.BlockSpec(shp, lambda i: (0,) * len(shp))
    row = pl.BlockSpec((bm, D), lambda i: (i, 0))
    return pl.pallas_call(
        _linear_kernel,
        grid=(S // bm,),
        in_specs=[row, full((D, D)), full((1, D))],
        out_specs=row,
        out_shape=jax.ShapeDtypeStruct((S, D), F32),
    )(x, W, b.reshape(1, D))


def _final_kernel(o0_ref, d1_ref, d2_ref, F0, F1, F2, fb, out_ref):
    acc = _dot(o0_ref[...], F0[...])
    acc = acc + _dot(d1_ref[...], F1[...])
    acc = acc + _dot(d2_ref[...], F2[...])
    out_ref[...] = acc + fb[...]


def _final_call(o0, d1, d2, fW, fb, bm):
    S, D = o0.shape
    full = lambda shp: pl.BlockSpec(shp, lambda i: (0,) * len(shp))
    row = pl.BlockSpec((bm, D), lambda i: (i, 0))
    return pl.pallas_call(
        _final_kernel,
        grid=(S // bm,),
        in_specs=[row, row, row,
                  full((D, D)), full((D, D)), full((D, D)), full((1, D))],
        out_specs=row,
        out_shape=jax.ShapeDtypeStruct((S, D), F32),
    )(o0, d1, d2, fW[:D], fW[D:2 * D], fW[2 * D:], fb.reshape(1, D))


def _band_strip(bm, lb):
    L = lb.shape[-1]
    P = 512
    u = jnp.arange(P)
    lbf = lb.astype(F32)
    prof = jax.nn.one_hot(jnp.clip(u - 64, 0, L - 1), L, dtype=F32) @ lbf
    v = jnp.where((u >= 64) & (u <= 128), prof - lbf[0], 0.0)
    X = jnp.broadcast_to(v, (bm, P)).reshape(-1)[: bm * (P - 1)]
    return X.reshape(bm, P - 1)[:, : bm + 128]


BLOCK_SIZES_ = (1, 4, 16)


def kernel(x, params):
    xb = x[0]
    S, D = xb.shape
    outs = []
    for level, bs in enumerate(BLOCK_SIZES_):
        if bs == 1:
            cur = xb
        else:
            ap = params["block_agg"][level - 1]
            cur = _blockagg_call(xb.reshape(S // bs, bs, D), ap["W"], ap["b"],
                                 bm=min(256, S // bs))
        Sl = cur.shape[0]
        bm = min(256, Sl)
        p = params["pred"][level]
        intents, feats, msgs = _pre_call(cur, p, params["mp"][level], bm)
        band = _band_strip(bm, p["lb"])
        agg = _attn_call(intents, feats, msgs, band, 32 // (2 ** level), bm)
        lo = _gate_call(cur, agg, params["mp"][level], bm)
        if bs > 1:
            dp = params["block_dist"][level - 1]
            dist = _linear_call(lo, dp["W"], dp["b"], bm)
            outs.append(jnp.repeat(dist, bs, axis=0))
        else:
            outs.append(lo)
    out = _final_call(outs[0], outs[1], outs[2], params["fW"], params["fb"],
                      bm=256)
    return out[None]


# speedup vs baseline: 1.5530x; 1.1574x over previous
import functools
import math

import jax
import jax.numpy as jnp
from jax.experimental import pallas as pl

NEG_INF = float("-inf")
F32 = jnp.float32
BF16 = jnp.bfloat16


def _dot(a, b):
    return jnp.dot(a.astype(BF16), b.astype(BF16), preferred_element_type=F32)


def _dotf(a, b):
    return jnp.dot(a, b, preferred_element_type=F32)


def _topk_softmax(s, kk):
    rowmax = jnp.max(s, axis=-1, keepdims=True)

    def body(_, t):
        m = jnp.max(t, axis=-1, keepdims=True)
        return jnp.where(t >= m, NEG_INF, t)

    t = jax.lax.fori_loop(0, kk - 1, body, s)
    thr = jnp.max(t, axis=-1, keepdims=True)
    w = jnp.where(s >= thr, jnp.exp(s - rowmax), 0.0)
    return w / jnp.sum(w, axis=-1, keepdims=True)


def _pre_kernel(x_ref, iW1, ib1, iW2, ib2, fW1, fb1, fW2, fb2, mW, mb,
                int_ref, feat_ref, msg_ref):
    x = x_ref[...]
    h = jax.nn.gelu(_dotf(x, iW1[...]) + ib1[...])
    int_ref[...] = _dotf(h, iW2[...]) + ib2[...]
    h = jax.nn.gelu(_dotf(x, fW1[...]) + fb1[...])
    feat_ref[...] = _dotf(h, fW2[...]) + fb2[...]
    msg_ref[...] = (_dot(x, mW[...]) + mb[...]).astype(BF16)


def _pre_call(cur, p, mp, bm):
    S, D = cur.shape
    H = D // 2
    full = lambda shp: pl.BlockSpec(shp, lambda i: (0,) * len(shp))
    row = pl.BlockSpec((bm, D), lambda i: (i, 0))
    return pl.pallas_call(
        _pre_kernel,
        grid=(S // bm,),
        in_specs=[row,
                  full((D, H)), full((1, H)), full((H, D)), full((1, D)),
                  full((D, H)), full((1, H)), full((H, D)), full((1, D)),
                  full((D, D)), full((1, D))],
        out_specs=(row, row, row),
        out_shape=(jax.ShapeDtypeStruct((S, D), F32),
                   jax.ShapeDtypeStruct((S, D), F32),
                   jax.ShapeDtypeStruct((S, D), BF16)),
    )(cur, p["iW1"], p["ib1"].reshape(1, H), p["iW2"], p["ib2"].reshape(1, D),
      p["fW1"], p["fb1"].reshape(1, H), p["fW2"], p["fb2"].reshape(1, D),
      mp["mW"].astype(BF16), mp["mb"].reshape(1, D))


def _attn_kernel(kk, scale, i0, int_ref, feat_ref, msg_ref, band_ref, out_ref):
    s = jax.lax.dot_general(int_ref[...], feat_ref[...],
                            (((1,), (1,)), ((), ())),
                            preferred_element_type=F32) * scale
    BM, W = s.shape
    BW = band_ref.shape[1]
    lo = max(0, i0 - (BW - BM))
    dlo = lo - (i0 - (BW - BM))
    sb = s[:, lo:] + band_ref[:, dlo:]
    s = jnp.concatenate([s[:, :lo], sb], axis=1) if lo > 0 else sb
    rows = i0 + jax.lax.broadcasted_iota(jnp.int32, (BM, W), 0)
    cols = jax.lax.broadcasted_iota(jnp.int32, (BM, W), 1)
    s = jnp.where(cols > rows, NEG_INF, s)
    p = _topk_softmax(s, kk)
    out_ref[...] = (_dot(p, msg_ref[...])).astype(BF16)


def _attn_call(intents, feats, msgs, band, kk, bm):
    S, D = intents.shape
    scale = 1.0 / math.sqrt(D)
    outs = []
    for i in range(S // bm):
        W = (i + 1) * bm
        out = pl.pallas_call(
            functools.partial(_attn_kernel, kk, scale, i * bm),
            grid=(1,),
            in_specs=[pl.BlockSpec((bm, D), lambda g, i=i: (i, 0)),
                      pl.BlockSpec((W, D), lambda g: (0, 0)),
                      pl.BlockSpec((W, D), lambda g: (0, 0)),
                      pl.BlockSpec(band.shape, lambda g: (0, 0))],
            out_specs=pl.BlockSpec((bm, D), lambda g: (0, 0)),
            out_shape=jax.ShapeDtypeStruct((bm, D), BF16),
        )(intents, feats, msgs, band)
        outs.append(out)
    return jnp.concatenate(outs, axis=0)


def _gate_kernel(x_ref, agg_ref, aW, ab, gWx, gWa, gb, oW, ob, out_ref):
    x = x_ref[...]
    a2 = _dot(agg_ref[...], aW[...]) + ab[...]
    g = jax.nn.sigmoid(_dot(x, gWx[...]) + _dot(a2, gWa[...]) + gb[...])
    u = x * (1.0 - g) + a2 * g
    out_ref[...] = (_dot(u, oW[...]) + ob[...]).astype(BF16)


def _gate_call(cur, agg, mp, bm):
    S, D = cur.shape
    full = lambda shp: pl.BlockSpec(shp, lambda i: (0,) * len(shp))
    row = pl.BlockSpec((bm, D), lambda i: (i, 0))
    return pl.pallas_call(
        _gate_kernel,
        grid=(S // bm,),
        in_specs=[row, row,
                  full((D, D)), full((1, D)),
                  full((D, D)), full((D, D)), full((1, D)),
                  full((D, D)), full((1, D))],
        out_specs=row,
        out_shape=jax.ShapeDtypeStruct((S, D), BF16),
    )(cur, agg, mp["aW"].astype(BF16), mp["ab"].reshape(1, D),
      mp["gW"][:D].astype(BF16), mp["gW"][D:].astype(BF16),
      mp["gb"].reshape(1, D),
      mp["oW"].astype(BF16), mp["ob"].reshape(1, D))


def _level_kernel(kk, scale, xr_ref, aggW, aggb, iW1, ib1, iW2, ib2,
                  fW1, fb1, fW2, fb2, mW, mb, aW, ab, gWx, gWa, gb,
                  oW, ob, dW, db, band_ref, dist_ref):
    cur = jax.nn.gelu(_dotf(jnp.mean(xr_ref[...], axis=1), aggW[...])
                      + aggb[...])
    hi = jax.nn.gelu(_dotf(cur, iW1[...]) + ib1[...])
    intents = _dotf(hi, iW2[...]) + ib2[...]
    hf = jax.nn.gelu(_dotf(cur, fW1[...]) + fb1[...])
    feats = _dotf(hf, fW2[...]) + fb2[...]
    msgs = _dot(cur, mW[...]) + mb[...]
    s = jax.lax.dot_general(intents, feats, (((1,), (1,)), ((), ())),
                            preferred_element_type=F32) * scale
    n = s.shape[0]
    s = s + band_ref[:, band_ref.shape[1] - n:]
    rows = jax.lax.broadcasted_iota(jnp.int32, (n, n), 0)
    cols = jax.lax.broadcasted_iota(jnp.int32, (n, n), 1)
    s = jnp.where(cols > rows, NEG_INF, s)
    p = _topk_softmax(s, kk)
    agg = _dot(p, msgs)
    a2 = _dot(agg, aW[...]) + ab[...]
    g = jax.nn.sigmoid(_dot(cur, gWx[...]) + _dot(a2, gWa[...]) + gb[...])
    u = cur * (1.0 - g) + a2 * g
    lo_ = _dot(u, oW[...]) + ob[...]
    dist_ref[...] = (_dot(lo_, dW[...]) + db[...]).astype(BF16)


def _level_call(xr, ap, p, mp, dp, kk):
    nb, bs, D = xr.shape
    H = D // 2
    scale = 1.0 / math.sqrt(D)
    band = _band_strip(nb, p["lb"])
    full = lambda shp: pl.BlockSpec(shp, lambda g: (0,) * len(shp))
    return pl.pallas_call(
        functools.partial(_level_kernel, kk, scale),
        grid=(1,),
        in_specs=[full((nb, bs, D)),
                  full((D, D)), full((1, D)),
                  full((D, H)), full((1, H)), full((H, D)), full((1, D)),
                  full((D, H)), full((1, H)), full((H, D)), full((1, D)),
                  full((D, D)), full((1, D)),
                  full((D, D)), full((1, D)),
                  full((D, D)), full((D, D)), full((1, D)),
                  full((D, D)), full((1, D)),
                  full((D, D)), full((1, D)),
                  full(band.shape)],
        out_specs=full((nb, D)),
        out_shape=jax.ShapeDtypeStruct((nb, D), BF16),
    )(xr, ap["W"], ap["b"].reshape(1, D),
      p["iW1"], p["ib1"].reshape(1, H), p["iW2"], p["ib2"].reshape(1, D),
      p["fW1"], p["fb1"].reshape(1, H), p["fW2"], p["fb2"].reshape(1, D),
      mp["mW"].astype(BF16), mp["mb"].reshape(1, D),
      mp["aW"].astype(BF16), mp["ab"].reshape(1, D),
      mp["gW"][:D].astype(BF16), mp["gW"][D:].astype(BF16),
      mp["gb"].reshape(1, D),
      mp["oW"].astype(BF16), mp["ob"].reshape(1, D),
      dp["W"].astype(BF16), dp["b"].reshape(1, D),
      band)


def _final_kernel(o0_ref, d1_ref, d2_ref, F0, F1, F2, fb, out_ref):
    acc = _dot(o0_ref[...], F0[...])
    acc = acc + _dot(d1_ref[...], F1[...])
    acc = acc + _dot(d2_ref[...], F2[...])
    out_ref[...] = acc + fb[...]


def _final_call(o0, d1, d2, fW, fb, bm):
    S, D = o0.shape
    full = lambda shp: pl.BlockSpec(shp, lambda i: (0,) * len(shp))
    row = pl.BlockSpec((bm, D), lambda i: (i, 0))
    fWc = fW.astype(BF16)
    return pl.pallas_call(
        _final_kernel,
        grid=(S // bm,),
        in_specs=[row, row, row,
                  full((D, D)), full((D, D)), full((D, D)), full((1, D))],
        out_specs=row,
        out_shape=jax.ShapeDtypeStruct((S, D), F32),
    )(o0, d1, d2, fWc[:D], fWc[D:2 * D], fWc[2 * D:], fb.reshape(1, D))


def _band_strip(bm, lb):
    L = lb.shape[-1]
    P = max(512, 2 * bm)
    u = jnp.arange(P)
    lbf = lb.astype(F32)
    prof = jax.nn.one_hot(jnp.clip(u - 64, 0, L - 1), L, dtype=F32) @ lbf
    v = jnp.where((u >= 64) & (u <= 128), prof - lbf[0], 0.0)
    X = jnp.broadcast_to(v, (bm, P)).reshape(-1)[: bm * (P - 1)]
    return X.reshape(bm, P - 1)[:, : bm + 128]


BLOCK_SIZES_ = (1, 4, 16)


def kernel(x, params):
    xb = x[0]
    S, D = xb.shape
    outs = []

    p = params["pred"][0]
    mp = params["mp"][0]
    intents, feats, msgs = _pre_call(xb, p, mp, 256)
    band = _band_strip(512, p["lb"])
    agg = _attn_call(intents, feats, msgs, band, 32, 512)
    outs.append(_gate_call(xb, agg, mp, 256))

    for level, bs in ((1, 4), (2, 16)):
        xr = xb.reshape(S // bs, bs, D)
        dist = _level_call(xr, params["block_agg"][level - 1],
                           params["pred"][level], params["mp"][level],
                           params["block_dist"][level - 1], 32 // (2 ** level))
        outs.append(jnp.repeat(dist, bs, axis=0))

    out = _final_call(outs[0], outs[1], outs[2], params["fW"], params["fb"],
                      bm=256)
    return out[None]


# speedup vs baseline: 1.5887x; 1.0230x over previous
import functools
import math

import jax
import jax.numpy as jnp
from jax.experimental import pallas as pl

NEG_INF = float("-inf")
F32 = jnp.float32
BF16 = jnp.bfloat16


def _dot(a, b):
    return jnp.dot(a.astype(BF16), b.astype(BF16), preferred_element_type=F32)


def _dotf(a, b):
    return jnp.dot(a, b, preferred_element_type=F32)


def _topk_softmax(s, kk):
    rowmax = jnp.max(s, axis=-1, keepdims=True)

    def body(_, t):
        m = jnp.max(t, axis=-1, keepdims=True)
        return jnp.where(t >= m, NEG_INF, t)

    t = jax.lax.fori_loop(0, kk - 1, body, s)
    thr = jnp.max(t, axis=-1, keepdims=True)
    w = jnp.where(s >= thr, jnp.exp(s - rowmax), 0.0)
    return w / jnp.sum(w, axis=-1, keepdims=True)


def _pre_kernel(x_ref, iW1, ib1, iW2, ib2, fW1, fb1, fW2, fb2, mW, mb,
                int_ref, feat_ref, msg_ref):
    x = x_ref[...]
    h = jax.nn.gelu(_dotf(x, iW1[...]) + ib1[...])
    int_ref[...] = _dotf(h, iW2[...]) + ib2[...]
    h = jax.nn.gelu(_dotf(x, fW1[...]) + fb1[...])
    feat_ref[...] = _dotf(h, fW2[...]) + fb2[...]
    msg_ref[...] = (_dot(x, mW[...]) + mb[...]).astype(BF16)


def _pre_call(cur, p, mp, bm):
    S, D = cur.shape
    H = D // 2
    full = lambda shp: pl.BlockSpec(shp, lambda i: (0,) * len(shp))
    row = pl.BlockSpec((bm, D), lambda i: (i, 0))
    return pl.pallas_call(
        _pre_kernel,
        grid=(S // bm,),
        in_specs=[row,
                  full((D, H)), full((1, H)), full((H, D)), full((1, D)),
                  full((D, H)), full((1, H)), full((H, D)), full((1, D)),
                  full((D, D)), full((1, D))],
        out_specs=(row, row, row),
        out_shape=(jax.ShapeDtypeStruct((S, D), F32),
                   jax.ShapeDtypeStruct((S, D), F32),
                   jax.ShapeDtypeStruct((S, D), BF16)),
    )(cur, p["iW1"], p["ib1"].reshape(1, H), p["iW2"], p["ib2"].reshape(1, D),
      p["fW1"], p["fb1"].reshape(1, H), p["fW2"], p["fb2"].reshape(1, D),
      mp["mW"].astype(BF16), mp["mb"].reshape(1, D))


def _attn_kernel(kk, scale, i0, int_ref, feat_ref, msg_ref, band_ref, out_ref):
    s = jax.lax.dot_general(int_ref[...], feat_ref[...],
                            (((1,), (1,)), ((), ())),
                            preferred_element_type=F32) * scale
    BM, W = s.shape
    BW = band_ref.shape[1]
    lo = max(0, i0 - (BW - BM))
    dlo = lo - (i0 - (BW - BM))
    sb = s[:, lo:] + band_ref[:, dlo:]
    s = jnp.concatenate([s[:, :lo], sb], axis=1) if lo > 0 else sb
    rows = i0 + jax.lax.broadcasted_iota(jnp.int32, (BM, W), 0)
    cols = jax.lax.broadcasted_iota(jnp.int32, (BM, W), 1)
    s = jnp.where(cols > rows, NEG_INF, s)
    p = _topk_softmax(s, kk)
    out_ref[...] = (_dot(p, msg_ref[...])).astype(BF16)


def _attn_call(intents, feats, msgs, band, kk, bm):
    S, D = intents.shape
    scale = 1.0 / math.sqrt(D)
    outs = []
    for i in range(S // bm):
        W = (i + 1) * bm
        out = pl.pallas_call(
            functools.partial(_attn_kernel, kk, scale, i * bm),
            grid=(1,),
            in_specs=[pl.BlockSpec((bm, D), lambda g, i=i: (i, 0)),
                      pl.BlockSpec((W, D), lambda g: (0, 0)),
                      pl.BlockSpec((W, D), lambda g: (0, 0)),
                      pl.BlockSpec(band.shape, lambda g: (0, 0))],
            out_specs=pl.BlockSpec((bm, D), lambda g: (0, 0)),
            out_shape=jax.ShapeDtypeStruct((bm, D), BF16),
        )(intents, feats, msgs, band)
        outs.append(out)
    return jnp.concatenate(outs, axis=0)


def _gate_kernel(x_ref, agg_ref, aW, ab, gWx, gWa, gb, oW, ob, out_ref):
    x = x_ref[...]
    a2 = _dot(agg_ref[...], aW[...]) + ab[...]
    g = jax.nn.sigmoid(_dot(x, gWx[...]) + _dot(a2, gWa[...]) + gb[...])
    u = x * (1.0 - g) + a2 * g
    out_ref[...] = (_dot(u, oW[...]) + ob[...]).astype(BF16)


def _gate_call(cur, agg, mp, bm):
    S, D = cur.shape
    full = lambda shp: pl.BlockSpec(shp, lambda i: (0,) * len(shp))
    row = pl.BlockSpec((bm, D), lambda i: (i, 0))
    return pl.pallas_call(
        _gate_kernel,
        grid=(S // bm,),
        in_specs=[row, row,
                  full((D, D)), full((1, D)),
                  full((D, D)), full((D, D)), full((1, D)),
                  full((D, D)), full((1, D))],
        out_specs=row,
        out_shape=jax.ShapeDtypeStruct((S, D), BF16),
    )(cur, agg, mp["aW"].astype(BF16), mp["ab"].reshape(1, D),
      mp["gW"][:D].astype(BF16), mp["gW"][D:].astype(BF16),
      mp["gb"].reshape(1, D),
      mp["oW"].astype(BF16), mp["ob"].reshape(1, D))


def _pre2_kernel(xr_ref, aggW, aggb, iW1, ib1, iW2, ib2, fW1, fb1, fW2, fb2,
                 mW, mb, cur_ref, int_ref, feat_ref, msg_ref):
    cur = jax.nn.gelu(_dotf(jnp.mean(xr_ref[...], axis=1), aggW[...])
                      + aggb[...])
    cur_ref[...] = cur
    h = jax.nn.gelu(_dotf(cur, iW1[...]) + ib1[...])
    int_ref[...] = _dotf(h, iW2[...]) + ib2[...]
    h = jax.nn.gelu(_dotf(cur, fW1[...]) + fb1[...])
    feat_ref[...] = _dotf(h, fW2[...]) + fb2[...]
    msg_ref[...] = (_dot(cur, mW[...]) + mb[...]).astype(BF16)


def _pre2_call(xr, ap, p, mp, bm):
    nb, bs, D = xr.shape
    H = D // 2
    full = lambda shp: pl.BlockSpec(shp, lambda i: (0,) * len(shp))
    row = pl.BlockSpec((bm, D), lambda i: (i, 0))
    return pl.pallas_call(
        _pre2_kernel,
        grid=(nb // bm,),
        in_specs=[pl.BlockSpec((bm, bs, D), lambda i: (i, 0, 0)),
                  full((D, D)), full((1, D)),
                  full((D, H)), full((1, H)), full((H, D)), full((1, D)),
                  full((D, H)), full((1, H)), full((H, D)), full((1, D)),
                  full((D, D)), full((1, D))],
        out_specs=(row, row, row, row),
        out_shape=(jax.ShapeDtypeStruct((nb, D), F32),
                   jax.ShapeDtypeStruct((nb, D), F32),
                   jax.ShapeDtypeStruct((nb, D), F32),
                   jax.ShapeDtypeStruct((nb, D), BF16)),
    )(xr, ap["W"], ap["b"].reshape(1, D),
      p["iW1"], p["ib1"].reshape(1, H), p["iW2"], p["ib2"].reshape(1, D),
      p["fW1"], p["fb1"].reshape(1, H), p["fW2"], p["fb2"].reshape(1, D),
      mp["mW"].astype(BF16), mp["mb"].reshape(1, D))


def _attngd_kernel(kk, scale, cur_ref, int_ref, feat_ref, msg_ref, band_ref,
                   aW, ab, gWx, gWa, gb, oW, ob, dW, db, dist_ref):
    cur = cur_ref[...]
    s = jax.lax.dot_general(int_ref[...], feat_ref[...],
                            (((1,), (1,)), ((), ())),
                            preferred_element_type=F32) * scale
    n = s.shape[0]
    s = s + band_ref[:, band_ref.shape[1] - n:]
    rows = jax.lax.broadcasted_iota(jnp.int32, (n, n), 0)
    cols = jax.lax.broadcasted_iota(jnp.int32, (n, n), 1)
    s = jnp.where(cols > rows, NEG_INF, s)
    p = _topk_softmax(s, kk)
    agg = _dot(p, msg_ref[...])
    a2 = _dot(agg, aW[...]) + ab[...]
    g = jax.nn.sigmoid(_dot(cur, gWx[...]) + _dot(a2, gWa[...]) + gb[...])
    u = cur * (1.0 - g) + a2 * g
    lo_ = _dot(u, oW[...]) + ob[...]
    dist_ref[...] = (_dot(lo_, dW[...]) + db[...]).astype(BF16)


def _attngd_call(cur, intents, feats, msgs, band, mp, dp, kk):
    nb, D = cur.shape
    scale = 1.0 / math.sqrt(D)
    full = lambda shp: pl.BlockSpec(shp, lambda g: (0,) * len(shp))
    return pl.pallas_call(
        functools.partial(_attngd_kernel, kk, scale),
        grid=(1,),
        in_specs=[full((nb, D)), full((nb, D)), full((nb, D)), full((nb, D)),
                  full(band.shape),
                  full((D, D)), full((1, D)),
                  full((D, D)), full((D, D)), full((1, D)),
                  full((D, D)), full((1, D)),
                  full((D, D)), full((1, D))],
        out_specs=full((nb, D)),
        out_shape=jax.ShapeDtypeStruct((nb, D), BF16),
    )(cur, intents, feats, msgs, band,
      mp["aW"].astype(BF16), mp["ab"].reshape(1, D),
      mp["gW"][:D].astype(BF16), mp["gW"][D:].astype(BF16),
      mp["gb"].reshape(1, D),
      mp["oW"].astype(BF16), mp["ob"].reshape(1, D),
      dp["W"].astype(BF16), dp["b"].reshape(1, D))


def _gatefinal_kernel(x_ref, agg_ref, d1_ref, d2_ref,
                      aW, ab, gWx, gWa, gb, oW, ob,
                      F0, F1, F2, fb, out_ref):
    x = x_ref[...]
    a2 = _dot(agg_ref[...], aW[...]) + ab[...]
    g = jax.nn.sigmoid(_dot(x, gWx[...]) + _dot(a2, gWa[...]) + gb[...])
    u = x * (1.0 - g) + a2 * g
    lo_ = _dot(u, oW[...]) + ob[...]
    acc = _dot(lo_, F0[...])
    acc = acc + _dot(d1_ref[...], F1[...])
    acc = acc + _dot(d2_ref[...], F2[...])
    out_ref[...] = acc + fb[...]


def _gatefinal_call(x, agg, d1, d2, mp, fW, fb, bm):
    S, D = x.shape
    full = lambda shp: pl.BlockSpec(shp, lambda i: (0,) * len(shp))
    row = pl.BlockSpec((bm, D), lambda i: (i, 0))
    fWc = fW.astype(BF16)
    return pl.pallas_call(
        _gatefinal_kernel,
        grid=(S // bm,),
        in_specs=[row, row, row, row,
                  full((D, D)), full((1, D)),
                  full((D, D)), full((D, D)), full((1, D)),
                  full((D, D)), full((1, D)),
                  full((D, D)), full((D, D)), full((D, D)), full((1, D))],
        out_specs=row,
        out_shape=jax.ShapeDtypeStruct((S, D), F32),
    )(x, agg, d1, d2,
      mp["aW"].astype(BF16), mp["ab"].reshape(1, D),
      mp["gW"][:D].astype(BF16), mp["gW"][D:].astype(BF16),
      mp["gb"].reshape(1, D),
      mp["oW"].astype(BF16), mp["ob"].reshape(1, D),
      fWc[:D], fWc[D:2 * D], fWc[2 * D:], fb.reshape(1, D))


def _band_strip(bm, lb):
    L = lb.shape[-1]
    P = max(512, 2 * bm)
    u = jnp.arange(P)
    lbf = lb.astype(F32)
    prof = jax.nn.one_hot(jnp.clip(u - 64, 0, L - 1), L, dtype=F32) @ lbf
    v = jnp.where((u >= 64) & (u <= 128), prof - lbf[0], 0.0)
    X = jnp.broadcast_to(v, (bm, P)).reshape(-1)[: bm * (P - 1)]
    return X.reshape(bm, P - 1)[:, : bm + 128]


BLOCK_SIZES_ = (1, 4, 16)


def kernel(x, params):
    xb = x[0]
    S, D = xb.shape

    dists = []
    for level, bs in ((1, 4), (2, 16)):
        nb = S // bs
        xr = xb.reshape(nb, bs, D)
        p = params["pred"][level]
        cur, intents, feats, msgs = _pre2_call(
            xr, params["block_agg"][level - 1], p, params["mp"][level],
            bm=nb // 2)
        band = _band_strip(nb, p["lb"])
        dist = _attngd_call(cur, intents, feats, msgs, band,
                            params["mp"][level],
                            params["block_dist"][level - 1],
                            32 // (2 ** level))
        dists.append(jnp.repeat(dist, bs, axis=0))

    p = params["pred"][0]
    mp = params["mp"][0]
    intents, feats, msgs = _pre_call(xb, p, mp, 256)
    band = _band_strip(512, p["lb"])
    agg = _attn_call(intents, feats, msgs, band, 32, 512)
    out = _gatefinal_call(xb, agg, dists[0], dists[1], mp,
                          params["fW"], params["fb"], bm=256)
    return out[None]


# speedup vs baseline: 1.6800x; 1.0575x over previous
import functools
import math

import jax
import jax.numpy as jnp
from jax.experimental import pallas as pl

NEG_INF = float("-inf")
F32 = jnp.float32
BF16 = jnp.bfloat16


def _dot(a, b):
    return jnp.dot(a.astype(BF16), b.astype(BF16), preferred_element_type=F32)


def _dotf(a, b):
    return jnp.dot(a, b, preferred_element_type=F32)


def _topk_softmax(s, kk):
    rowmax = jnp.max(s, axis=-1, keepdims=True)
    u = jax.lax.bitcast_convert_type(s, jnp.uint32)
    key = u ^ jnp.where(u >> 31 == 0, jnp.uint32(0x80000000),
                        jnp.uint32(0xFFFFFFFF))
    BM = s.shape[0]

    def body(i, t):
        cand = t | (jnp.uint32(1) << (jnp.uint32(31) - i.astype(jnp.uint32)))
        cnt = jnp.sum((key >= cand).astype(jnp.int32), axis=-1, keepdims=True)
        return jnp.where(cnt >= kk, cand, t)

    t = jax.lax.fori_loop(0, 32, body, jnp.zeros((BM, 1), jnp.uint32))
    w = jnp.where(key >= t, jnp.exp(s - rowmax), 0.0)
    return w / jnp.sum(w, axis=-1, keepdims=True)


def _pre_kernel(x_ref, iW1, ib1, iW2, ib2, fW1, fb1, fW2, fb2, mW, mb,
                int_ref, feat_ref, msg_ref):
    x = x_ref[...]
    h = jax.nn.gelu(_dotf(x, iW1[...]) + ib1[...])
    int_ref[...] = _dotf(h, iW2[...]) + ib2[...]
    h = jax.nn.gelu(_dotf(x, fW1[...]) + fb1[...])
    feat_ref[...] = _dotf(h, fW2[...]) + fb2[...]
    msg_ref[...] = (_dot(x, mW[...]) + mb[...]).astype(BF16)


def _pre_call(cur, p, mp, bm):
    S, D = cur.shape
    H = D // 2
    full = lambda shp: pl.BlockSpec(shp, lambda i: (0,) * len(shp))
    row = pl.BlockSpec((bm, D), lambda i: (i, 0))
    return pl.pallas_call(
        _pre_kernel,
        grid=(S // bm,),
        in_specs=[row,
                  full((D, H)), full((1, H)), full((H, D)), full((1, D)),
                  full((D, H)), full((1, H)), full((H, D)), full((1, D)),
                  full((D, D)), full((1, D))],
        out_specs=(row, row, row),
        out_shape=(jax.ShapeDtypeStruct((S, D), F32),
                   jax.ShapeDtypeStruct((S, D), F32),
                   jax.ShapeDtypeStruct((S, D), BF16)),
    )(cur, p["iW1"], p["ib1"].reshape(1, H), p["iW2"], p["ib2"].reshape(1, D),
      p["fW1"], p["fb1"].reshape(1, H), p["fW2"], p["fb2"].reshape(1, D),
      mp["mW"].astype(BF16), mp["mb"].reshape(1, D))


def _attn_kernel(kk, scale, i0, int_ref, feat_ref, msg_ref, band_ref, out_ref):
    s = jax.lax.dot_general(int_ref[...], feat_ref[...],
                            (((1,), (1,)), ((), ())),
                            preferred_element_type=F32) * scale
    BM, W = s.shape
    BW = band_ref.shape[1]
    lo = max(0, i0 - (BW - BM))
    dlo = lo - (i0 - (BW - BM))
    sb = s[:, lo:] + band_ref[:, dlo:]
    s = jnp.concatenate([s[:, :lo], sb], axis=1) if lo > 0 else sb
    rows = i0 + jax.lax.broadcasted_iota(jnp.int32, (BM, W), 0)
    cols = jax.lax.broadcasted_iota(jnp.int32, (BM, W), 1)
    s = jnp.where(cols > rows, NEG_INF, s)
    p = _topk_softmax(s, kk)
    out_ref[...] = (_dot(p, msg_ref[...])).astype(BF16)


def _attn_call(intents, feats, msgs, band, kk, bm):
    S, D = intents.shape
    scale = 1.0 / math.sqrt(D)
    outs = []
    for i in range(S // bm):
        W = (i + 1) * bm
        out = pl.pallas_call(
            functools.partial(_attn_kernel, kk, scale, i * bm),
            grid=(1,),
            in_specs=[pl.BlockSpec((bm, D), lambda g, i=i: (i, 0)),
                      pl.BlockSpec((W, D), lambda g: (0, 0)),
                      pl.BlockSpec((W, D), lambda g: (0, 0)),
                      pl.BlockSpec(band.shape, lambda g: (0, 0))],
            out_specs=pl.BlockSpec((bm, D), lambda g: (0, 0)),
            out_shape=jax.ShapeDtypeStruct((bm, D), BF16),
        )(intents, feats, msgs, band)
        outs.append(out)
    return jnp.concatenate(outs, axis=0)


def _gate_kernel(x_ref, agg_ref, aW, ab, gWx, gWa, gb, oW, ob, out_ref):
    x = x_ref[...]
    a2 = _dot(agg_ref[...], aW[...]) + ab[...]
    g = jax.nn.sigmoid(_dot(x, gWx[...]) + _dot(a2, gWa[...]) + gb[...])
    u = x * (1.0 - g) + a2 * g
    out_ref[...] = (_dot(u, oW[...]) + ob[...]).astype(BF16)


def _gate_call(cur, agg, mp, bm):
    S, D = cur.shape
    full = lambda shp: pl.BlockSpec(shp, lambda i: (0,) * len(shp))
    row = pl.BlockSpec((bm, D), lambda i: (i, 0))
    return pl.pallas_call(
        _gate_kernel,
        grid=(S // bm,),
        in_specs=[row, row,
                  full((D, D)), full((1, D)),
                  full((D, D)), full((D, D)), full((1, D)),
                  full((D, D)), full((1, D))],
        out_specs=row,
        out_shape=jax.ShapeDtypeStruct((S, D), BF16),
    )(cur, agg, mp["aW"].astype(BF16), mp["ab"].reshape(1, D),
      mp["gW"][:D].astype(BF16), mp["gW"][D:].astype(BF16),
      mp["gb"].reshape(1, D),
      mp["oW"].astype(BF16), mp["ob"].reshape(1, D))


def _pre2_kernel(xr_ref, aggW, aggb, iW1, ib1, iW2, ib2, fW1, fb1, fW2, fb2,
                 mW, mb, cur_ref, int_ref, feat_ref, msg_ref):
    cur = jax.nn.gelu(_dotf(jnp.mean(xr_ref[...], axis=1), aggW[...])
                      + aggb[...])
    cur_ref[...] = cur
    h = jax.nn.gelu(_dotf(cur, iW1[...]) + ib1[...])
    int_ref[...] = _dotf(h, iW2[...]) + ib2[...]
    h = jax.nn.gelu(_dotf(cur, fW1[...]) + fb1[...])
    feat_ref[...] = _dotf(h, fW2[...]) + fb2[...]
    msg_ref[...] = (_dot(cur, mW[...]) + mb[...]).astype(BF16)


def _pre2_call(xr, ap, p, mp, bm):
    nb, bs, D = xr.shape
    H = D // 2
    full = lambda shp: pl.BlockSpec(shp, lambda i: (0,) * len(shp))
    row = pl.BlockSpec((bm, D), lambda i: (i, 0))
    return pl.pallas_call(
        _pre2_kernel,
        grid=(nb // bm,),
        in_specs=[pl.BlockSpec((bm, bs, D), lambda i: (i, 0, 0)),
                  full((D, D)), full((1, D)),
                  full((D, H)), full((1, H)), full((H, D)), full((1, D)),
                  full((D, H)), full((1, H)), full((H, D)), full((1, D)),
                  full((D, D)), full((1, D))],
        out_specs=(row, row, row, row),
        out_shape=(jax.ShapeDtypeStruct((nb, D), F32),
                   jax.ShapeDtypeStruct((nb, D), F32),
                   jax.ShapeDtypeStruct((nb, D), F32),
                   jax.ShapeDtypeStruct((nb, D), BF16)),
    )(xr, ap["W"], ap["b"].reshape(1, D),
      p["iW1"], p["ib1"].reshape(1, H), p["iW2"], p["ib2"].reshape(1, D),
      p["fW1"], p["fb1"].reshape(1, H), p["fW2"], p["fb2"].reshape(1, D),
      mp["mW"].astype(BF16), mp["mb"].reshape(1, D))


def _attngd_kernel(kk, scale, cur_ref, int_ref, feat_ref, msg_ref, band_ref,
                   aW, ab, gWx, gWa, gb, oW, ob, dW, db, dist_ref):
    cur = cur_ref[...]
    s = jax.lax.dot_general(int_ref[...], feat_ref[...],
                            (((1,), (1,)), ((), ())),
                            preferred_element_type=F32) * scale
    n = s.shape[0]
    s = s + band_ref[:, band_ref.shape[1] - n:]
    rows = jax.lax.broadcasted_iota(jnp.int32, (n, n), 0)
    cols = jax.lax.broadcasted_iota(jnp.int32, (n, n), 1)
    s = jnp.where(cols > rows, NEG_INF, s)
    p = _topk_softmax(s, kk)
    agg = _dot(p, msg_ref[...])
    a2 = _dot(agg, aW[...]) + ab[...]
    g = jax.nn.sigmoid(_dot(cur, gWx[...]) + _dot(a2, gWa[...]) + gb[...])
    u = cur * (1.0 - g) + a2 * g
    lo_ = _dot(u, oW[...]) + ob[...]
    dist_ref[...] = (_dot(lo_, dW[...]) + db[...]).astype(BF16)


def _attngd_call(cur, intents, feats, msgs, band, mp, dp, kk):
    nb, D = cur.shape
    scale = 1.0 / math.sqrt(D)
    full = lambda shp: pl.BlockSpec(shp, lambda g: (0,) * len(shp))
    return pl.pallas_call(
        functools.partial(_attngd_kernel, kk, scale),
        grid=(1,),
        in_specs=[full((nb, D)), full((nb, D)), full((nb, D)), full((nb, D)),
                  full(band.shape),
                  full((D, D)), full((1, D)),
                  full((D, D)), full((D, D)), full((1, D)),
                  full((D, D)), full((1, D)),
                  full((D, D)), full((1, D))],
        out_specs=full((nb, D)),
        out_shape=jax.ShapeDtypeStruct((nb, D), BF16),
    )(cur, intents, feats, msgs, band,
      mp["aW"].astype(BF16), mp["ab"].reshape(1, D),
      mp["gW"][:D].astype(BF16), mp["gW"][D:].astype(BF16),
      mp["gb"].reshape(1, D),
      mp["oW"].astype(BF16), mp["ob"].reshape(1, D),
      dp["W"].astype(BF16), dp["b"].reshape(1, D))


def _gatefinal_kernel(x_ref, agg_ref, d1_ref, d2_ref,
                      aW, ab, gWx, gWa, gb, oW, ob,
                      F0, F1, F2, fb, out_ref):
    x = x_ref[...]
    a2 = _dot(agg_ref[...], aW[...]) + ab[...]
    g = jax.nn.sigmoid(_dot(x, gWx[...]) + _dot(a2, gWa[...]) + gb[...])
    u = x * (1.0 - g) + a2 * g
    lo_ = _dot(u, oW[...]) + ob[...]
    acc = _dot(lo_, F0[...])
    acc = acc + _dot(d1_ref[...], F1[...])
    acc = acc + _dot(d2_ref[...], F2[...])
    out_ref[...] = acc + fb[...]


def _gatefinal_call(x, agg, d1, d2, mp, fW, fb, bm):
    S, D = x.shape
    full = lambda shp: pl.BlockSpec(shp, lambda i: (0,) * len(shp))
    row = pl.BlockSpec((bm, D), lambda i: (i, 0))
    fWc = fW.astype(BF16)
    return pl.pallas_call(
        _gatefinal_kernel,
        grid=(S // bm,),
        in_specs=[row, row, row, row,
                  full((D, D)), full((1, D)),
                  full((D, D)), full((D, D)), full((1, D)),
                  full((D, D)), full((1, D)),
                  full((D, D)), full((D, D)), full((D, D)), full((1, D))],
        out_specs=row,
        out_shape=jax.ShapeDtypeStruct((S, D), F32),
    )(x, agg, d1, d2,
      mp["aW"].astype(BF16), mp["ab"].reshape(1, D),
      mp["gW"][:D].astype(BF16), mp["gW"][D:].astype(BF16),
      mp["gb"].reshape(1, D),
      mp["oW"].astype(BF16), mp["ob"].reshape(1, D),
      fWc[:D], fWc[D:2 * D], fWc[2 * D:], fb.reshape(1, D))


def _band_strip(bm, lb):
    L = lb.shape[-1]
    P = max(512, 2 * bm)
    u = jnp.arange(P)
    lbf = lb.astype(F32)
    prof = jax.nn.one_hot(jnp.clip(u - 64, 0, L - 1), L, dtype=F32) @ lbf
    v = jnp.where((u >= 64) & (u <= 128), prof - lbf[0], 0.0)
    X = jnp.broadcast_to(v, (bm, P)).reshape(-1)[: bm * (P - 1)]
    return X.reshape(bm, P - 1)[:, : bm + 128]


BLOCK_SIZES_ = (1, 4, 16)


def kernel(x, params):
    xb = x[0]
    S, D = xb.shape

    dists = []
    for level, bs in ((1, 4), (2, 16)):
        nb = S // bs
        xr = xb.reshape(nb, bs, D)
        p = params["pred"][level]
        cur, intents, feats, msgs = _pre2_call(
            xr, params["block_agg"][level - 1], p, params["mp"][level],
            bm=nb // 2)
        band = _band_strip(nb, p["lb"])
        dist = _attngd_call(cur, intents, feats, msgs, band,
                            params["mp"][level],
                            params["block_dist"][level - 1],
                            32 // (2 ** level))
        dists.append(jnp.repeat(dist, bs, axis=0))

    p = params["pred"][0]
    mp = params["mp"][0]
    intents, feats, msgs = _pre_call(xb, p, mp, 256)
    band = _band_strip(512, p["lb"])
    agg = _attn_call(intents, feats, msgs, band, 32, 512)
    out = _gatefinal_call(xb, agg, dists[0], dists[1], mp,
                          params["fW"], params["fb"], bm=256)
    return out[None]
